# baseline (speedup 1.0000x reference)
import functools

import jax
import jax.numpy as jnp
from jax.experimental import pallas as pl
from jax.experimental.pallas import tpu as pltpu

S, B, H, FF, E = 2048, 1, 1024, 2048, 8
TOP_P = 0.3

BS = 256
FB = 512


def _routing_weights(x, router_w):
    logits = x @ router_w.T
    probs = jax.nn.softmax(logits, axis=-1)
    sp = -jnp.sort(-probs, axis=-1)
    si = jnp.argsort(-probs, axis=-1)
    cum = jnp.cumsum(sp, axis=-1)
    t = jnp.argmax((cum > TOP_P).astype(jnp.int32), axis=-1)
    keep = jnp.arange(E)[None, :] <= t[:, None]
    w_sorted = jnp.where(keep, sp, 0.0)
    w = jnp.zeros((x.shape[0], E), jnp.float32)
    w = w.at[jnp.arange(x.shape[0])[:, None], si].set(w_sorted)
    return w


def _mlp_body(x_ref, g_ref, u_ref, d_ref, w_ref, o_ref):
    e = pl.program_id(1)
    f = pl.program_id(2)
    x = x_ref[...]
    g = jax.lax.dot_general(x, g_ref[0], (((1,), (1,)), ((), ())),
                            preferred_element_type=jnp.float32)
    u = jax.lax.dot_general(x, u_ref[0], (((1,), (1,)), ((), ())),
                            preferred_element_type=jnp.float32)
    h = (g * jax.nn.sigmoid(g)) * u
    y = jax.lax.dot_general(h, d_ref[0], (((1,), (1,)), ((), ())),
                            preferred_element_type=jnp.float32)
    contrib = w_ref[0, 0, :][:, None] * y

    @pl.when((e == 0) & (f == 0))
    def _():
        o_ref[...] = jnp.zeros_like(o_ref)

    o_ref[...] += contrib


def kernel(hidden_states, router_w, gate_w, up_w, down_w):
    s, b, h = hidden_states.shape
    x = hidden_states.reshape(s * b, h)
    w = _routing_weights(x, router_w)
    wT = w.T.reshape(E, 1, s * b)

    grid = (s * b // BS, E, FF // FB)
    out = pl.pallas_call(
        _mlp_body,
        grid=grid,
        in_specs=[
            pl.BlockSpec((BS, H), lambda i, e, f: (i, 0)),
            pl.BlockSpec((1, FB, H), lambda i, e, f: (e, f, 0)),
            pl.BlockSpec((1, FB, H), lambda i, e, f: (e, f, 0)),
            pl.BlockSpec((1, H, FB), lambda i, e, f: (e, 0, f)),
            pl.BlockSpec((1, 1, BS), lambda i, e, f: (e, 0, i)),
        ],
        out_specs=pl.BlockSpec((BS, H), lambda i, e, f: (i, 0)),
        out_shape=jax.ShapeDtypeStruct((s * b, h), jnp.float32),
        compiler_params=pltpu.CompilerParams(
            dimension_semantics=("parallel", "arbitrary", "arbitrary"),
        ),
    )(x, gate_w, up_w, down_w, wT)
    return out.reshape(s, b, h)
